# baseline (speedup 1.0000x reference)
import functools

import jax
import jax.numpy as jnp
from jax import lax
from jax.experimental import pallas as pl
from jax.experimental.pallas import tpu as pltpu
from jax.experimental.pallas import tpu_sc as plsc

N_REGIMES = 1000
EMBED_DIM = 128
BATCH = 16384

B_SC = 12288
B_TC = BATCH - B_SC

NUM_CORES = 2
NUM_SUBCORES = 16
NUM_WORKERS = NUM_CORES * NUM_SUBCORES
B_PER_W = B_SC // NUM_WORKERS
CHUNK = 128
N_CHUNKS = B_PER_W // CHUNK

TC_TILE = 1024
K_PAD = 1024


def _build_sc():
    mesh = plsc.VectorSubcoreMesh(core_axis_name="c", subcore_axis_name="s")

    @functools.partial(
        pl.kernel,
        mesh=mesh,
        out_type=jax.ShapeDtypeStruct((BATCH, EMBED_DIM), jnp.float32),
        scratch_types=[
            pltpu.VMEM((B_PER_W,), jnp.int32),
            pltpu.VMEM((B_PER_W, EMBED_DIM), jnp.float32),
            pltpu.VMEM_SHARED((N_REGIMES, EMBED_DIM), jnp.float32),
            pltpu.SemaphoreType.DMA,
            pltpu.SemaphoreType.DMA,
        ],
    )
    def gather_kernel(idx_hbm, table_hbm, out_hbm, idx_v, rows_v, table_sh,
                      gsem, wsem):
        s = lax.axis_index("s")
        wid = s * NUM_CORES + lax.axis_index("c")
        base = wid * B_PER_W
        idx_cp = pltpu.async_copy(idx_hbm.at[pl.ds(base, B_PER_W)], idx_v,
                                  wsem)
        @pl.when(s == 0)
        def _():
            pltpu.sync_copy(table_hbm, table_sh)
        plsc.subcore_barrier()
        idx_cp.wait()
        gathers = [
            pltpu.async_copy(
                table_sh.at[idx_v.at[pl.ds(c * CHUNK, CHUNK)]],
                rows_v.at[pl.ds(c * CHUNK, CHUNK)], gsem)
            for c in range(N_CHUNKS)
        ]
        writes = []
        for c in range(N_CHUNKS):
            gathers[c].wait()
            writes.append(
                pltpu.async_copy(
                    rows_v.at[pl.ds(c * CHUNK, CHUNK)],
                    out_hbm.at[pl.ds(base + c * CHUNK, CHUNK)], wsem))
        for w in writes:
            w.wait()

    return gather_kernel


def _tc_body(idx_ref, table_ref, out_ref):
    idx = idx_ref[...]
    iota = lax.broadcasted_iota(jnp.int32, (TC_TILE, K_PAD), 1)
    onehot = jnp.where(iota == idx[:, None], 1.0, 0.0)
    out_ref[...] = jnp.dot(onehot, table_ref[...],
                           preferred_element_type=jnp.float32)


_TC_LOOKUP = pl.pallas_call(
    _tc_body,
    grid=(B_TC // TC_TILE,),
    in_specs=[
        pl.BlockSpec((TC_TILE,), lambda i: (i,)),
        pl.BlockSpec((K_PAD, EMBED_DIM), lambda i: (0, 0)),
    ],
    out_specs=pl.BlockSpec((TC_TILE, EMBED_DIM), lambda i: (i, 0)),
    out_shape=jax.ShapeDtypeStruct((B_TC, EMBED_DIM), jnp.float32),
)

_SC_GATHER = _build_sc()


@jax.jit
def kernel(regime_ids, embedding_weight):
    ids = regime_ids.astype(jnp.int32)
    table_pad = jnp.pad(embedding_weight,
                        ((0, K_PAD - N_REGIMES), (0, 0)))
    out_full = _SC_GATHER(ids[:B_SC], embedding_weight)
    out_tc = _TC_LOOKUP(ids[B_SC:], table_pad)
    return lax.dynamic_update_slice(out_full, out_tc, (B_SC, 0))


# speedup vs baseline: 1.1299x; 1.1299x over previous
import functools

import jax
import jax.numpy as jnp
from jax import lax
from jax.experimental import pallas as pl
from jax.experimental.pallas import tpu as pltpu
from jax.experimental.pallas import tpu_sc as plsc

N_REGIMES = 1000
EMBED_DIM = 128
BATCH = 16384

NUM_CORES = 2
NUM_SUBCORES = 16
NUM_WORKERS = NUM_CORES * NUM_SUBCORES
B_PER_W = BATCH // NUM_WORKERS
CHUNK = 128
N_CHUNKS = B_PER_W // CHUNK


def _build():
    mesh = plsc.VectorSubcoreMesh(core_axis_name="c", subcore_axis_name="s")

    @functools.partial(
        pl.kernel,
        mesh=mesh,
        out_type=jax.ShapeDtypeStruct((BATCH, EMBED_DIM), jnp.float32),
        scratch_types=[
            pltpu.VMEM((B_PER_W,), jnp.int32),
            pltpu.VMEM((B_PER_W, EMBED_DIM), jnp.float32),
            pltpu.VMEM_SHARED((N_REGIMES, EMBED_DIM), jnp.float32),
            pltpu.SemaphoreType.DMA,
            pltpu.SemaphoreType.DMA,
        ],
    )
    def gather_kernel(idx_hbm, table_hbm, out_hbm, idx_v, rows_v, table_sh,
                      gsem, wsem):
        s = lax.axis_index("s")
        wid = s * NUM_CORES + lax.axis_index("c")
        base = wid * B_PER_W
        idx_cp = pltpu.async_copy(idx_hbm.at[pl.ds(base, B_PER_W)], idx_v,
                                  wsem)
        @pl.when(s < 15)
        def _():
            off = s * 64
            pltpu.sync_copy(table_hbm.at[pl.ds(off, 64)],
                            table_sh.at[pl.ds(off, 64)])
        @pl.when(s == 15)
        def _():
            pltpu.sync_copy(table_hbm.at[pl.ds(960, 40)],
                            table_sh.at[pl.ds(960, 40)])
        plsc.subcore_barrier()
        idx_cp.wait()
        gathers = [
            pltpu.async_copy(
                table_sh.at[idx_v.at[pl.ds(c * CHUNK, CHUNK)]],
                rows_v.at[pl.ds(c * CHUNK, CHUNK)], gsem)
            for c in range(N_CHUNKS)
        ]
        writes = []
        for c in range(N_CHUNKS):
            gathers[c].wait()
            writes.append(
                pltpu.async_copy(
                    rows_v.at[pl.ds(c * CHUNK, CHUNK)],
                    out_hbm.at[pl.ds(base + c * CHUNK, CHUNK)], wsem))
        for w in writes:
            w.wait()

    return gather_kernel


_GATHER = _build()


@jax.jit
def kernel(regime_ids, embedding_weight):
    return _GATHER(regime_ids.astype(jnp.int32), embedding_weight)
